# baseline (speedup 1.0000x reference)
import jax
import jax.numpy as jnp
from jax.experimental import pallas as pl

_BETA = 0.25


def _vq_block(z_ref, e_ref, zq_ref, loss_ref):
    zct = z_ref[0]
    e = e_ref[...]
    n = e.shape[0]
    zsq = jnp.sum(zct * zct, axis=0, keepdims=True)
    esq = jnp.sum(e * e, axis=1)
    mm = jax.lax.dot_general(e, zct, (((1,), (0,)), ((), ())),
                             preferred_element_type=jnp.float32)
    dist = (zsq + esq[:, None]) - 2.0 * mm
    dmin = jnp.min(dist, axis=0, keepdims=True)
    iota = jax.lax.broadcasted_iota(jnp.int32, dist.shape, 0)
    idx = jnp.min(jnp.where(dist == dmin, iota, jnp.int32(n)),
                  axis=0)
    onehot = (iota == idx[None, :]).astype(jnp.float32)
    zq = jax.lax.dot_general(e, onehot, (((0,), (0,)), ((), ())),
                             preferred_element_type=jnp.float32,
                             precision=jax.lax.Precision.HIGHEST)
    zq_ref[...] = zq[None]
    diff = zq - zct
    loss_ref[...] = jnp.full((1, 1, 128), jnp.sum(diff * diff), jnp.float32)


def kernel(z, emb_weight):
    B, C, H, W = z.shape
    N, D = emb_weight.shape
    T = H * W
    bt = 256
    tb = T // bt
    zr = z.reshape(B, C, T)

    zq_r, loss_parts = pl.pallas_call(
        _vq_block,
        grid=(B, tb),
        in_specs=[
            pl.BlockSpec((1, C, bt), lambda i, j: (i, 0, j)),
            pl.BlockSpec((N, D), lambda i, j: (0, 0)),
        ],
        out_specs=[
            pl.BlockSpec((1, C, bt), lambda i, j: (i, 0, j)),
            pl.BlockSpec((1, 1, 128), lambda i, j: (i * tb + j, 0, 0)),
        ],
        out_shape=[
            jax.ShapeDtypeStruct((B, C, T), jnp.float32),
            jax.ShapeDtypeStruct((B * tb, 1, 128), jnp.float32),
        ],
    )(zr, emb_weight)

    sq_sum = jnp.sum(loss_parts[:, 0, 0])
    mean_sq = sq_sum / (B * T * D)
    loss = _BETA * mean_sq + mean_sq
    return (zq_r.reshape(B, C, H, W), loss)


# speedup vs baseline: 1.8041x; 1.8041x over previous
import jax
import jax.numpy as jnp
from jax.experimental import pallas as pl

_BETA = 0.25


def _vq_block(z_ref, e_ref, ehi_ref, emid_ref, zq_ref, loss_ref):
    zct = z_ref[0]
    e = e_ref[...]
    n = e.shape[0]
    zsq = jnp.sum(zct * zct, axis=0, keepdims=True)
    esq = jnp.sum(e * e, axis=1)
    mm = jax.lax.dot_general(e, zct, (((1,), (0,)), ((), ())),
                             preferred_element_type=jnp.float32)
    dist = (zsq + esq[:, None]) - 2.0 * mm
    dmin = jnp.min(dist, axis=0, keepdims=True)
    iota = jax.lax.broadcasted_iota(jnp.int32, dist.shape, 0)
    idx = jnp.min(jnp.where(dist == dmin, iota, jnp.int32(n)),
                  axis=0)
    onehot = (iota == idx[None, :]).astype(jnp.bfloat16)
    zq = (jax.lax.dot_general(ehi_ref[...], onehot, (((0,), (0,)), ((), ())),
                              preferred_element_type=jnp.float32)
          + jax.lax.dot_general(emid_ref[...], onehot, (((0,), (0,)), ((), ())),
                                preferred_element_type=jnp.float32))
    zq_ref[...] = zq[None]
    loss_ref[...] = jnp.full((1, 1, 128), jnp.sum(dmin), jnp.float32)


def kernel(z, emb_weight):
    B, C, H, W = z.shape
    N, D = emb_weight.shape
    T = H * W
    zr = z.reshape(B, C, T)
    e_hi = emb_weight.astype(jnp.bfloat16)
    e_mid = (emb_weight - e_hi.astype(jnp.float32)).astype(jnp.bfloat16)

    zq_r, loss_parts = pl.pallas_call(
        _vq_block,
        grid=(B,),
        in_specs=[
            pl.BlockSpec((1, C, T), lambda i: (i, 0, 0)),
            pl.BlockSpec((N, D), lambda i: (0, 0)),
            pl.BlockSpec((N, D), lambda i: (0, 0)),
            pl.BlockSpec((N, D), lambda i: (0, 0)),
        ],
        out_specs=[
            pl.BlockSpec((1, C, T), lambda i: (i, 0, 0)),
            pl.BlockSpec((1, 1, 128), lambda i: (i, 0, 0)),
        ],
        out_shape=[
            jax.ShapeDtypeStruct((B, C, T), jnp.float32),
            jax.ShapeDtypeStruct((B, 1, 128), jnp.float32),
        ],
    )(zr, emb_weight, e_hi, e_mid)

    sq_sum = jnp.sum(loss_parts[:, 0, 0])
    mean_sq = sq_sum / (B * T * D)
    loss = _BETA * mean_sq + mean_sq
    return (zq_r.reshape(B, C, H, W), loss)


# speedup vs baseline: 1.9822x; 1.0987x over previous
import jax
import jax.numpy as jnp
from jax.experimental import pallas as pl
from jax.experimental.pallas import tpu as pltpu

_BETA = 0.25


def _vq_block(z_ref, e_ref, zq_ref, loss_ref, acc_ref):
    i = pl.program_id(0)
    nsteps = pl.num_programs(0)
    zct = z_ref[0]
    e = e_ref[...]
    n = e.shape[0]
    zsq = jnp.sum(zct * zct, axis=0, keepdims=True)
    esq = jnp.sum(e * e, axis=1)
    mm = jax.lax.dot_general(e, zct, (((1,), (0,)), ((), ())),
                             preferred_element_type=jnp.float32)
    dist = (zsq + esq[:, None]) - 2.0 * mm
    dmin = jnp.min(dist, axis=0, keepdims=True)
    iota = jax.lax.broadcasted_iota(jnp.int32, dist.shape, 0)
    idx = jnp.min(jnp.where(dist == dmin, iota, jnp.int32(n)),
                  axis=0)
    onehot = (iota == idx[None, :]).astype(jnp.bfloat16)
    e_hi = e.astype(jnp.bfloat16)
    e_mid = (e - e_hi.astype(jnp.float32)).astype(jnp.bfloat16)
    zq = (jax.lax.dot_general(e_hi, onehot, (((0,), (0,)), ((), ())),
                              preferred_element_type=jnp.float32)
          + jax.lax.dot_general(e_mid, onehot, (((0,), (0,)), ((), ())),
                                preferred_element_type=jnp.float32))
    zq_ref[...] = zq[None]

    part = jnp.sum(dmin)

    @pl.when(i == 0)
    def _init():
        acc_ref[0, 0] = part

    @pl.when(i > 0)
    def _acc():
        acc_ref[0, 0] += part

    @pl.when(i == nsteps - 1)
    def _fin():
        total = acc_ref[0, 0]
        denom = zq_ref.shape[1] * zq_ref.shape[2] * nsteps
        mean_sq = total / denom
        loss_ref[...] = jnp.full((1, 128), (1.0 + _BETA) * mean_sq,
                                 jnp.float32)


def kernel(z, emb_weight):
    B, C, H, W = z.shape
    N, D = emb_weight.shape
    T = H * W
    zr = z.reshape(B, C, T)

    zq_r, loss_out = pl.pallas_call(
        _vq_block,
        grid=(B,),
        in_specs=[
            pl.BlockSpec((1, C, T), lambda i: (i, 0, 0)),
            pl.BlockSpec((N, D), lambda i: (0, 0)),
        ],
        out_specs=[
            pl.BlockSpec((1, C, T), lambda i: (i, 0, 0)),
            pl.BlockSpec((1, 128), lambda i: (0, 0)),
        ],
        out_shape=[
            jax.ShapeDtypeStruct((B, C, T), jnp.float32),
            jax.ShapeDtypeStruct((1, 128), jnp.float32),
        ],
        scratch_shapes=[pltpu.SMEM((1, 1), jnp.float32)],
    )(zr, emb_weight)

    return (zq_r.reshape(B, C, H, W), loss_out[0, 0])
